# baseline (speedup 1.0000x reference)
import dataclasses
import functools

import jax
import jax.numpy as jnp
from jax import lax
from jax.experimental import pallas as pl
from jax.experimental.pallas import tpu as pltpu
from jax.experimental.pallas import tpu_sc as plsc

_NC, _NS, _L = 2, 16, 16
_NW = _NC * _NS
_B = 16384
_R = 196
_K = 512
_BW = _B // _NW
_CH = 32
_NCHUNK = _BW // _CH
_KFULL = _K // _L
_IDXH = _B // 2
_DMAX = (_R + _NW - 1) // _NW
_COLS = _B // _NW
_HC = _COLS // 2


def _mesh():
  return plsc.VectorSubcoreMesh(core_axis_name="c", subcore_axis_name="s",
                                num_cores=_NC, num_subcores=_NS)


def _cparams():
  cp = pltpu.CompilerParams()
  if "needs_layout_passes" in pltpu.CompilerParams.__dataclass_fields__:
    cp = dataclasses.replace(cp, needs_layout_passes=False)
  return cp


def _tree_sum(vals):
  vals = list(vals)
  while len(vals) > 1:
    nxt = [a + b for a, b in zip(vals[0::2], vals[1::2])]
    if len(vals) % 2:
      nxt.append(vals[-1])
    vals = nxt
  return vals[0]


def _scale_vec(norm):
  return jnp.where(norm > 1.0, 1.0 / (norm + 1e-7),
                   jnp.ones((_L,), jnp.float32))


def _make_k1_body(n):
  ha = n // 2
  hb = n - ha

  def _k1_body(idx_hbm, wrt_hbm, outt_hbm, idx_v, bufa_v, bufb_v, gout_v,
               sem_a, sem_b):
    wid = lax.axis_index("s") * _NC + lax.axis_index("c")
    lane = lax.iota(jnp.int32, _L)

    def desc_a(d):
      return pltpu.make_async_copy(wrt_hbm.at[d].at[0], bufa_v, sem_a)

    def desc_b(d):
      return pltpu.make_async_copy(wrt_hbm.at[d].at[1], bufb_v, sem_b)

    desc_a(wid).start()
    desc_b(wid).start()
    pltpu.sync_copy(idx_hbm, idx_v)

    def pass_a(h):
      @plsc.parallel_loop(0, _IDXH // _L, unroll=8)
      def _pa(w):
        iv = idx_v[pl.ds(h * _IDXH + w * _L, _L)]
        gout_v[pl.ds(w * _L, _L)] = plsc.load_gather(
            bufa_v, [jnp.minimum(iv, ha - 1)])

    def pass_b(h):
      @plsc.parallel_loop(0, _IDXH // _L, unroll=8)
      def _pb(w):
        iv = idx_v[pl.ds(h * _IDXH + w * _L, _L)]
        g = plsc.load_gather(bufb_v, [jnp.clip(iv - ha, 0, hb - 1)])
        plsc.store_scatter(gout_v, [w * _L + lane], g, mask=iv >= ha)

    @pl.loop(0, _DMAX)
    def _rows(j):
      d = wid + j * _NW

      @pl.when(d < _R)
      def _():
        dn = d + _NW
        with jax.named_scope("k1_wait_a"):
          desc_a(d).wait()
        pass_a(0)
        with jax.named_scope("k1_wait_b"):
          desc_b(d).wait()
        pass_b(0)
        pltpu.sync_copy(gout_v, outt_hbm.at[d, pl.ds(0, _IDXH)])
        pass_a(1)

        @pl.when(dn < _R)
        def _():
          desc_a(dn).start()
        pass_b(1)

        @pl.when(dn < _R)
        def _():
          desc_b(dn).start()
        pltpu.sync_copy(gout_v, outt_hbm.at[d, pl.ds(_IDXH, _IDXH)])

  return _k1_body


def _k2_body(idx_hbm, wk_hbm, outtr_hbm, outt_hbm, outk_hbm,
             idx_v, slab_v, buf_k0, buf_k1, obuf_k0, obuf_k1,
             sem_s, gsem_k0, gsem_k1, osem_k0, osem_k1):
  wid = lax.axis_index("s") * _NC + lax.axis_index("c")
  base = wid * _BW
  pltpu.sync_copy(idx_hbm.at[pl.ds(base, _BW)], idx_v)

  bufs_k = (buf_k0, buf_k1)
  obufs_k = (obuf_k0, obuf_k1)
  gsems_k = (gsem_k0, gsem_k1)
  osems_k = (osem_k0, osem_k1)

  slab0 = pltpu.make_async_copy(
      outtr_hbm.at[:, pl.ds(base, _HC)], slab_v, sem_s)
  slab0.start()

  def gather_desc(cc):
    p = cc % 2
    s = idx_v.at[pl.ds(cc * _CH, _CH)]
    return pltpu.make_async_copy(wk_hbm.at[s], bufs_k[p], gsems_k[p])

  def out_desc(cc):
    p = cc % 2
    d = pl.ds(base + cc * _CH, _CH)
    return pltpu.make_async_copy(obufs_k[p], outk_hbm.at[d], osems_k[p])

  def compute(p):
    buf_k, obuf_k = bufs_k[p], obufs_k[p]

    @plsc.parallel_loop(0, _CH)
    def _row_k(r):
      wins = [buf_k[r, pl.ds(j * _L, _L)] for j in range(_KFULL)]
      norm = jnp.broadcast_to(
          jnp.sum(_tree_sum([jnp.abs(w) for w in wins])), (_L,))
      scale = _scale_vec(norm)
      for j in range(_KFULL):
        obuf_k[r, pl.ds(j * _L, _L)] = wins[j] * scale

  gather_desc(0).start()
  gather_desc(1).start()
  for cc in range(_NCHUNK):
    gather_desc(cc).wait()
    if cc >= 2:
      out_desc(cc - 2).wait()
    compute(cc % 2)
    out_desc(cc).start()
    if cc + 2 < _NCHUNK:
      gather_desc(cc + 2).start()
  out_desc(_NCHUNK - 2).wait()
  out_desc(_NCHUNK - 1).wait()

  nch = _HC // _L

  for h in range(2):
    b0 = base + h * _HC
    if h == 0:
      slab0.wait()
    else:
      pltpu.async_copy(outtr_hbm.at[:, pl.ds(b0, _HC)], slab_v, sem_s).wait()

    def nacc(dd, accs):
      return tuple(accs[c] + jnp.abs(slab_v[dd, pl.ds(c * _L, _L)])
                   for c in range(nch))
    norms = lax.fori_loop(
        0, _R, nacc, tuple(jnp.zeros((_L,), jnp.float32) for _ in range(nch)),
        unroll=4)
    scales = [_scale_vec(n) for n in norms]

    @plsc.parallel_loop(0, _R, unroll=2)
    def _scale_rows(dd):
      for c in range(nch):
        slab_v[dd, pl.ds(c * _L, _L)] = (
            slab_v[dd, pl.ds(c * _L, _L)] * scales[c])

    pltpu.async_copy(slab_v, outt_hbm.at[:, pl.ds(b0, _HC)], sem_s).wait()


@jax.jit
def _run(inds, W_region, W_kernel):
  n = W_region.shape[0]
  wrt = W_region.T

  k1 = pl.kernel(
      _make_k1_body(n),
      out_type=jax.ShapeDtypeStruct((_R, _B), jnp.float32),
      mesh=_mesh(),
      scratch_types=[
          pltpu.VMEM((_B,), jnp.int32),
          pltpu.VMEM((n // 2,), jnp.float32),
          pltpu.VMEM((n - n // 2,), jnp.float32),
          pltpu.VMEM((_IDXH,), jnp.float32),
          pltpu.SemaphoreType.DMA,
          pltpu.SemaphoreType.DMA,
      ],
      compiler_params=_cparams(),
  )
  out_raw = k1(inds, wrt.reshape(_R, 2, n // 2))

  k2 = pl.kernel(
      _k2_body,
      out_type=(
          jax.ShapeDtypeStruct((_R, _B), jnp.float32),
          jax.ShapeDtypeStruct((_B, _K), jnp.float32),
      ),
      mesh=_mesh(),
      scratch_types=(
          [pltpu.VMEM((_BW,), jnp.int32),
           pltpu.VMEM((_R, _HC), jnp.float32)]
          + [pltpu.VMEM((_CH, _K), jnp.float32)] * 4
          + [pltpu.SemaphoreType.DMA] * 5
      ),
      compiler_params=_cparams(),
  )
  out_t, out_k = k2(inds, W_kernel, out_raw)
  return out_t.T, out_k


def kernel(inds, W_region, W_kernel):
  return _run(inds.astype(jnp.int32), W_region, W_kernel)


# speedup vs baseline: 2.1903x; 2.1903x over previous
import dataclasses
import functools

import jax
import jax.numpy as jnp
from jax import lax
from jax.experimental import pallas as pl
from jax.experimental.pallas import tpu as pltpu
from jax.experimental.pallas import tpu_sc as plsc

_NC, _NS, _L = 2, 16, 16
_NW = _NC * _NS
_B = 16384
_R = 196
_K = 512
_BW = _B // _NW
_CH = 32
_NCHUNK = _BW // _CH
_KFULL = _K // _L
_IDXH = _B // 2
_DMAX = (_R + _NW - 1) // _NW
_COLS = _B // _NW
_HC = _COLS // 2


def _mesh():
  return plsc.VectorSubcoreMesh(core_axis_name="c", subcore_axis_name="s",
                                num_cores=_NC, num_subcores=_NS)


def _cparams():
  cp = pltpu.CompilerParams()
  if "needs_layout_passes" in pltpu.CompilerParams.__dataclass_fields__:
    cp = dataclasses.replace(cp, needs_layout_passes=False)
  return cp


def _tree_sum(vals):
  vals = list(vals)
  while len(vals) > 1:
    nxt = [a + b for a, b in zip(vals[0::2], vals[1::2])]
    if len(vals) % 2:
      nxt.append(vals[-1])
    vals = nxt
  return vals[0]


def _scale_vec(norm):
  return jnp.where(norm > 1.0, 1.0 / (norm + 1e-7),
                   jnp.ones((_L,), jnp.float32))


def _k1_body(idx_hbm, wrt_hbm, outt_hbm, idx_v, row_v, gout_v, sem):
  wid = lax.axis_index("s") * _NC + lax.axis_index("c")
  pltpu.sync_copy(idx_hbm, idx_v)

  @pl.loop(0, _DMAX)
  def _rows(j):
    d = wid + j * _NW

    @pl.when(d < _R)
    def _():
      pltpu.async_copy(wrt_hbm.at[d], row_v, sem).wait()
      for h in range(2):
        @plsc.parallel_loop(0, _IDXH // _L, unroll=8)
        def _gather(w):
          iv = idx_v[pl.ds(h * _IDXH + w * _L, _L)]
          gout_v[pl.ds(w * _L, _L)] = plsc.load_gather(row_v, [iv])
        pltpu.sync_copy(gout_v, outt_hbm.at[d, pl.ds(h * _IDXH, _IDXH)])


def _k2_body(idx_hbm, wk_hbm, outtr_hbm, outt_hbm, outk_hbm,
             idx_v, slab_v, buf_k0, buf_k1, obuf_k0, obuf_k1,
             sem_s, gsem_k0, gsem_k1, osem_k0, osem_k1):
  wid = lax.axis_index("s") * _NC + lax.axis_index("c")
  base = wid * _BW
  pltpu.sync_copy(idx_hbm.at[pl.ds(base, _BW)], idx_v)

  bufs_k = (buf_k0, buf_k1)
  obufs_k = (obuf_k0, obuf_k1)
  gsems_k = (gsem_k0, gsem_k1)
  osems_k = (osem_k0, osem_k1)

  slab0 = pltpu.make_async_copy(
      outtr_hbm.at[:, pl.ds(base, _HC)], slab_v, sem_s)
  slab0.start()

  def gather_desc(cc):
    p = cc % 2
    s = idx_v.at[pl.ds(cc * _CH, _CH)]
    return pltpu.make_async_copy(wk_hbm.at[s], bufs_k[p], gsems_k[p])

  def out_desc(cc):
    p = cc % 2
    d = pl.ds(base + cc * _CH, _CH)
    return pltpu.make_async_copy(obufs_k[p], outk_hbm.at[d], osems_k[p])

  def compute(p):
    buf_k, obuf_k = bufs_k[p], obufs_k[p]

    @plsc.parallel_loop(0, _CH)
    def _row_k(r):
      wins = [buf_k[r, pl.ds(j * _L, _L)] for j in range(_KFULL)]
      norm = jnp.broadcast_to(
          jnp.sum(_tree_sum([jnp.abs(w) for w in wins])), (_L,))
      scale = _scale_vec(norm)
      for j in range(_KFULL):
        obuf_k[r, pl.ds(j * _L, _L)] = wins[j] * scale

  gather_desc(0).start()
  gather_desc(1).start()
  for cc in range(_NCHUNK):
    gather_desc(cc).wait()
    if cc >= 2:
      out_desc(cc - 2).wait()
    compute(cc % 2)
    out_desc(cc).start()
    if cc + 2 < _NCHUNK:
      gather_desc(cc + 2).start()
  out_desc(_NCHUNK - 2).wait()
  out_desc(_NCHUNK - 1).wait()

  nch = _HC // _L

  for h in range(2):
    b0 = base + h * _HC
    if h == 0:
      slab0.wait()
    else:
      pltpu.async_copy(outtr_hbm.at[:, pl.ds(b0, _HC)], slab_v, sem_s).wait()

    def nacc(dd, accs):
      return tuple(accs[c] + jnp.abs(slab_v[dd, pl.ds(c * _L, _L)])
                   for c in range(nch))
    norms = lax.fori_loop(
        0, _R, nacc, tuple(jnp.zeros((_L,), jnp.float32) for _ in range(nch)),
        unroll=4)
    scales = [_scale_vec(n) for n in norms]

    @plsc.parallel_loop(0, _R, unroll=2)
    def _scale_rows(dd):
      for c in range(nch):
        slab_v[dd, pl.ds(c * _L, _L)] = (
            slab_v[dd, pl.ds(c * _L, _L)] * scales[c])

    pltpu.async_copy(slab_v, outt_hbm.at[:, pl.ds(b0, _HC)], sem_s).wait()


@jax.jit
def _run(inds, W_region, W_kernel):
  n = W_region.shape[0]
  wrt = W_region.T

  k1 = pl.kernel(
      _k1_body,
      out_type=jax.ShapeDtypeStruct((_R, _B), jnp.float32),
      mesh=_mesh(),
      scratch_types=[
          pltpu.VMEM((_B,), jnp.int32),
          pltpu.VMEM((n,), jnp.float32),
          pltpu.VMEM((_IDXH,), jnp.float32),
          pltpu.SemaphoreType.DMA,
      ],
      compiler_params=_cparams(),
  )
  out_raw = k1(inds, wrt)

  k2 = pl.kernel(
      _k2_body,
      out_type=(
          jax.ShapeDtypeStruct((_R, _B), jnp.float32),
          jax.ShapeDtypeStruct((_B, _K), jnp.float32),
      ),
      mesh=_mesh(),
      scratch_types=(
          [pltpu.VMEM((_BW,), jnp.int32),
           pltpu.VMEM((_R, _HC), jnp.float32)]
          + [pltpu.VMEM((_CH, _K), jnp.float32)] * 4
          + [pltpu.SemaphoreType.DMA] * 5
      ),
      compiler_params=_cparams(),
  )
  out_t, out_k = k2(inds, W_kernel, out_raw)
  return out_t.T, out_k


def kernel(inds, W_region, W_kernel):
  return _run(inds.astype(jnp.int32), W_region, W_kernel)


# speedup vs baseline: 2.2109x; 1.0094x over previous
import dataclasses
import functools

import jax
import jax.numpy as jnp
from jax import lax
from jax.experimental import pallas as pl
from jax.experimental.pallas import tpu as pltpu
from jax.experimental.pallas import tpu_sc as plsc

_NC, _NS, _L = 2, 16, 16
_NW = _NC * _NS
_B = 16384
_R = 196
_K = 512
_BW = _B // _NW
_CH = 32
_NCHUNK = _BW // _CH
_KFULL = _K // _L
_IDXH = _B // 2
_DMAX = (_R + _NW - 1) // _NW
_COLS = _B // _NW
_HC = _COLS // 2


def _mesh():
  return plsc.VectorSubcoreMesh(core_axis_name="c", subcore_axis_name="s",
                                num_cores=_NC, num_subcores=_NS)


def _cparams():
  cp = pltpu.CompilerParams()
  if "needs_layout_passes" in pltpu.CompilerParams.__dataclass_fields__:
    cp = dataclasses.replace(cp, needs_layout_passes=False)
  return cp


def _tree_sum(vals):
  vals = list(vals)
  while len(vals) > 1:
    nxt = [a + b for a, b in zip(vals[0::2], vals[1::2])]
    if len(vals) % 2:
      nxt.append(vals[-1])
    vals = nxt
  return vals[0]


def _scale_vec(norm):
  return jnp.where(norm > 1.0, 1.0 / (norm + 1e-7),
                   jnp.ones((_L,), jnp.float32))


def _k1_body(idx_hbm, wrt_hbm, outt_hbm, idx_v, row_v, gout_v, sem):
  wid = lax.axis_index("s") * _NC + lax.axis_index("c")
  pltpu.sync_copy(idx_hbm, idx_v)

  @pl.loop(0, _DMAX)
  def _rows(j):
    d = wid + j * _NW

    @pl.when(d < _R)
    def _():
      pltpu.async_copy(wrt_hbm.at[d], row_v, sem).wait()
      for h in range(2):
        @plsc.parallel_loop(0, _IDXH // _L, unroll=8)
        def _gather(w):
          iv = idx_v[pl.ds(h * _IDXH + w * _L, _L)]
          gout_v[pl.ds(w * _L, _L)] = plsc.load_gather(row_v, [iv])
        pltpu.sync_copy(gout_v, outt_hbm.at[d, pl.ds(h * _IDXH, _IDXH)])


def _k2_body(idx_hbm, wk_hbm, outtr_hbm, outt_hbm, outk_hbm,
             idx_v, slab_v, buf_k0, buf_k1, obuf_k0, obuf_k1,
             sem_s, gsem_k0, gsem_k1, osem_k0, osem_k1):
  wid = lax.axis_index("s") * _NC + lax.axis_index("c")
  base = wid * _BW
  pltpu.sync_copy(idx_hbm.at[pl.ds(base, _BW)], idx_v)

  bufs_k = (buf_k0, buf_k1)
  obufs_k = (obuf_k0, obuf_k1)
  gsems_k = (gsem_k0, gsem_k1)
  osems_k = (osem_k0, osem_k1)

  slab0 = pltpu.make_async_copy(
      outtr_hbm.at[:, pl.ds(base, _HC)], slab_v, sem_s)
  slab0.start()

  nch = _HC // _L

  def renorm_slab():
    def nacc(dd, accs):
      return tuple(accs[c] + jnp.abs(slab_v[dd, pl.ds(c * _L, _L)])
                   for c in range(nch))
    norms = lax.fori_loop(
        0, _R, nacc, tuple(jnp.zeros((_L,), jnp.float32) for _ in range(nch)))
    scales = [_scale_vec(n) for n in norms]

    @pl.loop(0, _R)
    def _scale_rows(dd):
      for c in range(nch):
        slab_v[dd, pl.ds(c * _L, _L)] = (
            slab_v[dd, pl.ds(c * _L, _L)] * scales[c])

  slab0.wait()
  renorm_slab()
  pltpu.async_copy(slab_v, outt_hbm.at[:, pl.ds(base, _HC)], sem_s).wait()
  slab1 = pltpu.make_async_copy(
      outtr_hbm.at[:, pl.ds(base + _HC, _HC)], slab_v, sem_s)
  slab1.start()

  def gather_desc(cc):
    p = cc % 2
    s = idx_v.at[pl.ds(cc * _CH, _CH)]
    return pltpu.make_async_copy(wk_hbm.at[s], bufs_k[p], gsems_k[p])

  def out_desc(cc):
    p = cc % 2
    d = pl.ds(base + cc * _CH, _CH)
    return pltpu.make_async_copy(obufs_k[p], outk_hbm.at[d], osems_k[p])

  def compute(p):
    buf_k, obuf_k = bufs_k[p], obufs_k[p]

    @pl.loop(0, _CH)
    def _row_k(r):
      wins = [buf_k[r, pl.ds(j * _L, _L)] for j in range(_KFULL)]
      norm = jnp.broadcast_to(
          jnp.sum(_tree_sum([jnp.abs(w) for w in wins])), (_L,))
      scale = _scale_vec(norm)
      for j in range(_KFULL):
        obuf_k[r, pl.ds(j * _L, _L)] = wins[j] * scale

  gather_desc(0).start()
  gather_desc(1).start()
  for cc in range(_NCHUNK):
    gather_desc(cc).wait()
    if cc >= 2:
      out_desc(cc - 2).wait()
    compute(cc % 2)
    out_desc(cc).start()
    if cc + 2 < _NCHUNK:
      gather_desc(cc + 2).start()
  out_desc(_NCHUNK - 2).wait()
  out_desc(_NCHUNK - 1).wait()

  slab1.wait()
  renorm_slab()
  pltpu.async_copy(
      slab_v, outt_hbm.at[:, pl.ds(base + _HC, _HC)], sem_s).wait()


@jax.jit
def _run(inds, W_region, W_kernel):
  n = W_region.shape[0]
  wrt = W_region.T

  k1 = pl.kernel(
      _k1_body,
      out_type=jax.ShapeDtypeStruct((_R, _B), jnp.float32),
      mesh=_mesh(),
      scratch_types=[
          pltpu.VMEM((_B,), jnp.int32),
          pltpu.VMEM((n,), jnp.float32),
          pltpu.VMEM((_IDXH,), jnp.float32),
          pltpu.SemaphoreType.DMA,
      ],
      compiler_params=_cparams(),
  )
  out_raw = k1(inds, wrt)

  k2 = pl.kernel(
      _k2_body,
      out_type=(
          jax.ShapeDtypeStruct((_R, _B), jnp.float32),
          jax.ShapeDtypeStruct((_B, _K), jnp.float32),
      ),
      mesh=_mesh(),
      scratch_types=(
          [pltpu.VMEM((_BW,), jnp.int32),
           pltpu.VMEM((_R, _HC), jnp.float32)]
          + [pltpu.VMEM((_CH, _K), jnp.float32)] * 4
          + [pltpu.SemaphoreType.DMA] * 5
      ),
      compiler_params=_cparams(),
  )
  out_t, out_k = k2(inds, W_kernel, out_raw)
  return out_t.T, out_k


def kernel(inds, W_region, W_kernel):
  return _run(inds.astype(jnp.int32), W_region, W_kernel)


# speedup vs baseline: 2.2372x; 1.0119x over previous
import dataclasses
import functools

import jax
import jax.numpy as jnp
from jax import lax
from jax.experimental import pallas as pl
from jax.experimental.pallas import tpu as pltpu
from jax.experimental.pallas import tpu_sc as plsc

_NC, _NS, _L = 2, 16, 16
_NW = _NC * _NS
_B = 16384
_R = 196
_K = 512
_BW = _B // _NW
_CH = 32
_NCHUNK = _BW // _CH
_KFULL = _K // _L
_IDXH = _B // 2
_DMAX = (_R + _NW - 1) // _NW
_COLS = _B // _NW
_HC = _COLS // 2


def _mesh():
  return plsc.VectorSubcoreMesh(core_axis_name="c", subcore_axis_name="s",
                                num_cores=_NC, num_subcores=_NS)


def _cparams():
  cp = pltpu.CompilerParams()
  if "needs_layout_passes" in pltpu.CompilerParams.__dataclass_fields__:
    cp = dataclasses.replace(cp, needs_layout_passes=False)
  return cp


def _tree_sum(vals):
  vals = list(vals)
  while len(vals) > 1:
    nxt = [a + b for a, b in zip(vals[0::2], vals[1::2])]
    if len(vals) % 2:
      nxt.append(vals[-1])
    vals = nxt
  return vals[0]


def _scale_vec(norm):
  return jnp.where(norm > 1.0, 1.0 / (norm + 1e-7),
                   jnp.ones((_L,), jnp.float32))


def _k1_body(idx_hbm, wrt_hbm, outt_hbm, idx_v, row_v, gout_v, sem):
  wid = lax.axis_index("s") * _NC + lax.axis_index("c")
  pltpu.sync_copy(idx_hbm, idx_v)

  @pl.loop(0, _DMAX)
  def _rows(j):
    d = wid + j * _NW

    @pl.when(d < _R)
    def _():
      pltpu.async_copy(wrt_hbm.at[d], row_v, sem).wait()
      for h in range(2):
        @plsc.parallel_loop(0, _IDXH // _L, unroll=8)
        def _gather(w):
          iv = idx_v[pl.ds(h * _IDXH + w * _L, _L)]
          gout_v[pl.ds(w * _L, _L)] = plsc.load_gather(row_v, [iv])
        pltpu.sync_copy(gout_v, outt_hbm.at[d, pl.ds(h * _IDXH, _IDXH)])


def _k2_body(idx_hbm, wk_hbm, outtr_hbm, outt_hbm, outk_hbm,
             idx_v, slab_v, buf_k0, buf_k1, obuf_k0, obuf_k1,
             sem_s, gsem_k0, gsem_k1, osem_k0, osem_k1):
  wid = lax.axis_index("s") * _NC + lax.axis_index("c")
  base = wid * _BW
  pltpu.sync_copy(idx_hbm.at[pl.ds(base, _BW)], idx_v)

  bufs_k = (buf_k0, buf_k1)
  obufs_k = (obuf_k0, obuf_k1)
  gsems_k = (gsem_k0, gsem_k1)
  osems_k = (osem_k0, osem_k1)

  slab0 = pltpu.make_async_copy(
      outtr_hbm.at[:, pl.ds(base, _HC)], slab_v, sem_s)
  slab0.start()

  nch = _HC // _L

  def renorm_slab():
    def nacc(dd, accs):
      return tuple(accs[c] + jnp.abs(slab_v[dd, pl.ds(c * _L, _L)])
                   for c in range(nch))
    norms = lax.fori_loop(
        0, _R, nacc, tuple(jnp.zeros((_L,), jnp.float32) for _ in range(nch)))
    scales = [_scale_vec(n) for n in norms]

    @pl.loop(0, _R)
    def _scale_rows(dd):
      for c in range(nch):
        slab_v[dd, pl.ds(c * _L, _L)] = (
            slab_v[dd, pl.ds(c * _L, _L)] * scales[c])

  def gather_desc(cc):
    p = cc % 2
    s = idx_v.at[pl.ds(cc * _CH, _CH)]
    return pltpu.make_async_copy(wk_hbm.at[s], bufs_k[p], gsems_k[p])

  def out_desc(cc):
    p = cc % 2
    d = pl.ds(base + cc * _CH, _CH)
    return pltpu.make_async_copy(obufs_k[p], outk_hbm.at[d], osems_k[p])

  gather_desc(0).start()
  gather_desc(1).start()

  slab0.wait()
  renorm_slab()
  pltpu.async_copy(slab_v, outt_hbm.at[:, pl.ds(base, _HC)], sem_s).wait()
  slab1 = pltpu.make_async_copy(
      outtr_hbm.at[:, pl.ds(base + _HC, _HC)], slab_v, sem_s)
  slab1.start()

  def compute(p):
    buf_k, obuf_k = bufs_k[p], obufs_k[p]

    @pl.loop(0, _CH)
    def _row_k(r):
      wins = [buf_k[r, pl.ds(j * _L, _L)] for j in range(_KFULL)]
      norm = jnp.broadcast_to(
          jnp.sum(_tree_sum([jnp.abs(w) for w in wins])), (_L,))
      scale = _scale_vec(norm)
      for j in range(_KFULL):
        obuf_k[r, pl.ds(j * _L, _L)] = wins[j] * scale

  for cc in range(_NCHUNK):
    gather_desc(cc).wait()
    if cc >= 2:
      out_desc(cc - 2).wait()
    compute(cc % 2)
    out_desc(cc).start()
    if cc + 2 < _NCHUNK:
      gather_desc(cc + 2).start()
  out_desc(_NCHUNK - 2).wait()
  out_desc(_NCHUNK - 1).wait()

  slab1.wait()
  renorm_slab()
  pltpu.async_copy(
      slab_v, outt_hbm.at[:, pl.ds(base + _HC, _HC)], sem_s).wait()


@jax.jit
def _run(inds, W_region, W_kernel):
  n = W_region.shape[0]
  wrt = W_region.T

  k1 = pl.kernel(
      _k1_body,
      out_type=jax.ShapeDtypeStruct((_R, _B), jnp.float32),
      mesh=_mesh(),
      scratch_types=[
          pltpu.VMEM((_B,), jnp.int32),
          pltpu.VMEM((n,), jnp.float32),
          pltpu.VMEM((_IDXH,), jnp.float32),
          pltpu.SemaphoreType.DMA,
      ],
      compiler_params=_cparams(),
  )
  out_raw = k1(inds, wrt)

  k2 = pl.kernel(
      _k2_body,
      out_type=(
          jax.ShapeDtypeStruct((_R, _B), jnp.float32),
          jax.ShapeDtypeStruct((_B, _K), jnp.float32),
      ),
      mesh=_mesh(),
      scratch_types=(
          [pltpu.VMEM((_BW,), jnp.int32),
           pltpu.VMEM((_R, _HC), jnp.float32)]
          + [pltpu.VMEM((_CH, _K), jnp.float32)] * 4
          + [pltpu.SemaphoreType.DMA] * 5
      ),
      compiler_params=_cparams(),
  )
  out_t, out_k = k2(inds, W_kernel, out_raw)
  return out_t.T, out_k


def kernel(inds, W_region, W_kernel):
  return _run(inds.astype(jnp.int32), W_region, W_kernel)
